# baseline (speedup 1.0000x reference)
import jax
import jax.numpy as jnp
from jax.experimental import pallas as pl
from jax.experimental.pallas import tpu as pltpu

B, N, M, DIM = 16, 2048, 2048, 3
TI = 256
NI = N // TI


BB = 4
GS = B // BB


def _chamfer_body(p1_ref, p2_ref, out_ref, rowacc_ref, colacc_ref):
    g = pl.program_id(0)

    rs = None
    cs = None
    for bb in range(BB):
        p2 = p2_ref[bb]
        sq2 = jnp.sum(p2 * p2, axis=1, keepdims=True)
        one2 = jnp.ones_like(sq2)
        sq2_hi = sq2.astype(jnp.bfloat16).astype(jnp.float32)
        rhs = jnp.concatenate(
            [p2, one2, one2, sq2_hi, sq2 - sq2_hi],
            axis=1).astype(jnp.bfloat16)

        cm = None
        for it in range(NI):
            p1 = p1_ref[bb, it * TI:(it + 1) * TI, :]
            sq1 = jnp.sum(p1 * p1, axis=1, keepdims=True)
            one1 = jnp.ones_like(sq1)
            sq1_hi = sq1.astype(jnp.bfloat16).astype(jnp.float32)
            lhs = jnp.concatenate(
                [p1 * (-2.0), sq1_hi, sq1 - sq1_hi, one1, one1],
                axis=1).astype(jnp.bfloat16)
            d = jax.lax.dot_general(
                lhs, rhs, (((1,), (1,)), ((), ())),
                preferred_element_type=jnp.float32)
            rm = jnp.maximum(jnp.min(d, axis=1), 0.0)[None, :]
            rs = rm if rs is None else rs + rm
            cmt = jnp.min(d, axis=0, keepdims=True)
            cm = cmt if cm is None else jnp.minimum(cm, cmt)
        cmc = jnp.maximum(cm, 0.0)
        cs = cmc if cs is None else cs + cmc

    first = g == 0
    prev_row = jnp.where(first, jnp.zeros_like(rowacc_ref[...]),
                         rowacc_ref[...])
    rowacc_ref[...] = prev_row + rs
    prev_col = jnp.where(first, jnp.zeros_like(colacc_ref[...]),
                         colacc_ref[...])
    colacc_ref[...] = prev_col + cs

    @pl.when(g == GS - 1)
    def _final():
        total = (jnp.sum(rowacc_ref[...]) * (1.0 / (B * N))
                 + jnp.sum(colacc_ref[...]) * (1.0 / (B * M)))
        out_ref[...] = jnp.full_like(out_ref, total)


def kernel(pcs1, pcs2):
    out = pl.pallas_call(
        _chamfer_body,
        grid=(GS,),
        in_specs=[
            pl.BlockSpec((BB, N, DIM), lambda g: (g, 0, 0)),
            pl.BlockSpec((BB, M, DIM), lambda g: (g, 0, 0)),
        ],
        out_specs=pl.BlockSpec((1, 1), lambda g: (0, 0)),
        out_shape=jax.ShapeDtypeStruct((1, 1), jnp.float32),
        scratch_shapes=[
            pltpu.VMEM((1, TI), jnp.float32),
            pltpu.VMEM((1, M), jnp.float32),
        ],
        compiler_params=pltpu.CompilerParams(
            dimension_semantics=("arbitrary",)),
    )(pcs1, pcs2)
    return out[0, 0]


# speedup vs baseline: 1.5744x; 1.5744x over previous
import jax
import jax.numpy as jnp
from jax.experimental import pallas as pl
from jax.experimental.pallas import tpu as pltpu

B, N, M, DIM = 16, 2048, 2048, 3
TI = 256
NI = N // TI


BB = 4
GS = B // BB


def _chamfer_body(p1t_ref, p2t_ref, out_ref, rowacc_ref, colacc_ref):
    g = pl.program_id(0)

    rs = None
    cs = None
    for bb in range(BB):
        p2t = p2t_ref[bb]
        sq2 = jnp.sum(p2t * p2t, axis=0, keepdims=True)
        one2 = jnp.ones_like(sq2)
        sq2_hi = sq2.astype(jnp.bfloat16).astype(jnp.float32)
        rhs = jnp.concatenate(
            [p2t, one2, one2, sq2_hi, sq2 - sq2_hi],
            axis=0).astype(jnp.bfloat16)

        cm = None
        for it in range(NI):
            p1t = p1t_ref[bb, :, it * TI:(it + 1) * TI]
            sq1 = jnp.sum(p1t * p1t, axis=0, keepdims=True)
            one1 = jnp.ones_like(sq1)
            sq1_hi = sq1.astype(jnp.bfloat16).astype(jnp.float32)
            lhs = jnp.concatenate(
                [p1t * (-2.0), sq1_hi, sq1 - sq1_hi, one1, one1],
                axis=0).astype(jnp.bfloat16)
            d = jax.lax.dot_general(
                lhs, rhs, (((0,), (0,)), ((), ())),
                preferred_element_type=jnp.float32)
            rm = jnp.maximum(jnp.min(d, axis=1), 0.0)[None, :]
            rs = rm if rs is None else rs + rm
            cmt = jnp.min(d, axis=0, keepdims=True)
            cm = cmt if cm is None else jnp.minimum(cm, cmt)
        cmc = jnp.maximum(cm, 0.0)
        cs = cmc if cs is None else cs + cmc

    first = g == 0
    prev_row = jnp.where(first, jnp.zeros_like(rowacc_ref[...]),
                         rowacc_ref[...])
    rowacc_ref[...] = prev_row + rs
    prev_col = jnp.where(first, jnp.zeros_like(colacc_ref[...]),
                         colacc_ref[...])
    colacc_ref[...] = prev_col + cs

    @pl.when(g == GS - 1)
    def _final():
        total = (jnp.sum(rowacc_ref[...]) * (1.0 / (B * N))
                 + jnp.sum(colacc_ref[...]) * (1.0 / (B * M)))
        out_ref[...] = jnp.full_like(out_ref, total)


def kernel(pcs1, pcs2):
    p1t = jnp.transpose(pcs1, (0, 2, 1))
    p2t = jnp.transpose(pcs2, (0, 2, 1))
    out = pl.pallas_call(
        _chamfer_body,
        grid=(GS,),
        in_specs=[
            pl.BlockSpec((BB, DIM, N), lambda g: (g, 0, 0)),
            pl.BlockSpec((BB, DIM, M), lambda g: (g, 0, 0)),
        ],
        out_specs=pl.BlockSpec((1, 1), lambda g: (0, 0)),
        out_shape=jax.ShapeDtypeStruct((1, 1), jnp.float32),
        scratch_shapes=[
            pltpu.VMEM((1, TI), jnp.float32),
            pltpu.VMEM((1, M), jnp.float32),
        ],
        compiler_params=pltpu.CompilerParams(
            dimension_semantics=("arbitrary",)),
    )(p1t, p2t)
    return out[0, 0]
